# Initial kernel scaffold; baseline (speedup 1.0000x reference)
import jax
import jax.numpy as jnp
from jax.experimental import pallas as pl


def kernel(box_preds, cls_preds):
    raise NotImplementedError("write your pallas kernel here")


# speedup vs baseline: 80.6186x; 80.6186x over previous
import functools

import jax
import jax.numpy as jnp
from jax import lax
from jax.experimental import pallas as pl
from jax.experimental.pallas import tpu as pltpu

N = 20000
NUM_CLASS = 3
PRE_MAX = 4096
POST_MAX = 500
SCORE_THRESH = 0.1
NMS_THRESH = 0.5

B = 128
NB = PRE_MAX // B


def _iou_tile(x1c, x2c, y1c, y2c, ac, x1r, x2r, y1r, y2r, ar):
    iw = jnp.maximum(jnp.minimum(x2c, x2r) - jnp.maximum(x1c, x1r), 0.0)
    ih = jnp.maximum(jnp.minimum(y2c, y2r) - jnp.maximum(y1c, y1r), 0.0)
    inter = iw * ih
    union = ac + ar - inter
    iou = inter / jnp.maximum(union, 1e-6)
    return (iou > NMS_THRESH).astype(jnp.float32)


def _nms_body(x1r, x2r, y1r, y2r, ar, x1c, x2c, y1c, y2c, ac, valid, keep_ref):
    keep_ref[...] = valid[...]
    tri = (lax.broadcasted_iota(jnp.int32, (B, B), 0)
           < lax.broadcasted_iota(jnp.int32, (B, B), 1)).astype(jnp.float32)

    def block_step(b, _):
        cx1 = x1c[b]
        cx2 = x2c[b]
        cy1 = y1c[b]
        cy2 = y2c[b]
        ca = ac[b]
        rx1 = x1r[pl.ds(b, 1), :]
        rx2 = x2r[pl.ds(b, 1), :]
        ry1 = y1r[pl.ds(b, 1), :]
        ry2 = y2r[pl.ds(b, 1), :]
        ra = ar[pl.ds(b, 1), :]
        s_bb = _iou_tile(cx1, cx2, cy1, cy2, ca, rx1, rx2, ry1, ry2, ra) * tri
        v = keep_ref[pl.ds(b, 1), :]

        def cond(c):
            return c[1]

        def body(c):
            k = c[0]
            kn = v * (jnp.dot(k, s_bb, preferred_element_type=jnp.float32)
                      == 0.0).astype(jnp.float32)
            return kn, jnp.any(kn != k)

        k, _ = lax.while_loop(cond, body, (v, True))
        keep_ref[pl.ds(b, 1), :] = k

        def cross_step(cb, _):
            rx1 = x1r[pl.ds(cb, 1), :]
            rx2 = x2r[pl.ds(cb, 1), :]
            ry1 = y1r[pl.ds(cb, 1), :]
            ry2 = y2r[pl.ds(cb, 1), :]
            ra = ar[pl.ds(cb, 1), :]
            s = _iou_tile(cx1, cx2, cy1, cy2, ca, rx1, rx2, ry1, ry2, ra)
            supp = jnp.dot(k, s, preferred_element_type=jnp.float32)
            keep_ref[pl.ds(cb, 1), :] = (keep_ref[pl.ds(cb, 1), :]
                                         * (supp == 0.0).astype(jnp.float32))
            return 0

        lax.fori_loop(b + 1, NB, cross_step, 0)
        return 0

    lax.fori_loop(0, NB, block_step, 0)


_nms_call = pl.pallas_call(
    _nms_body,
    out_shape=jax.ShapeDtypeStruct((NB, B), jnp.float32),
)


def kernel(box_preds, cls_preds):
    rank_scores = jnp.max(cls_preds, axis=-1)
    scores = jax.nn.sigmoid(rank_scores)
    top_scores, top_idx = lax.top_k(scores, PRE_MAX)
    top_boxes = box_preds[top_idx]

    c = jnp.abs(jnp.cos(top_boxes[:, 6]))
    s = jnp.abs(jnp.sin(top_boxes[:, 6]))
    dx = jnp.abs(top_boxes[:, 3])
    dy = jnp.abs(top_boxes[:, 4])
    hx = 0.5 * (dx * c + dy * s)
    hy = 0.5 * (dx * s + dy * c)
    x1 = top_boxes[:, 0] - hx
    x2 = top_boxes[:, 0] + hx
    y1 = top_boxes[:, 1] - hy
    y2 = top_boxes[:, 1] + hy
    area = (x2 - x1) * (y2 - y1)

    x1r = x1.reshape(NB, B)
    x2r = x2.reshape(NB, B)
    y1r = y1.reshape(NB, B)
    y2r = y2.reshape(NB, B)
    ar = area.reshape(NB, B)
    valid = (top_scores > SCORE_THRESH).astype(jnp.float32).reshape(NB, B)

    keep = _nms_call(x1r, x2r, y1r, y2r, ar,
                     x1r[:, :, None], x2r[:, :, None], y1r[:, :, None],
                     y2r[:, :, None], ar[:, :, None], valid)
    keepb = keep.reshape(PRE_MAX) > 0.5

    sel_scores = jnp.where(keepb, top_scores, -1.0)
    final_scores, sel = lax.top_k(sel_scores, POST_MAX)
    final_boxes = top_boxes[sel]
    return jnp.concatenate([final_boxes, final_scores[:, None]], axis=-1)
